# baseline (speedup 1.0000x reference)
import functools

import jax
import jax.numpy as jnp
from jax.experimental import pallas as pl
from jax.experimental.pallas import tpu as pltpu

_H = 384
_W = 384
_SUB = 2
_HS = _H // _SUB
_WS = _W // _SUB
_NPAD = 1024
_RB = 256
_KC = 512
_NEV = 3 * _NPAD


def _match_body(b_ref, bpT_ref, sc_ref, spr_ref,
                outA_ref, outBv_ref, outBb_ref, outC_ref, macc_ref):
    step = pl.program_id(0)
    rows = b_ref[...]
    ax1 = rows[:, 0:1]
    ay1 = rows[:, 1:2]
    ax2 = rows[:, 2:3]
    ay2 = rows[:, 3:4]
    bx1 = bpT_ref[0:1, :]
    by1 = bpT_ref[1:2, :]
    bx2 = bpT_ref[2:3, :]
    by2 = bpT_ref[3:4, :]
    xl = jnp.maximum(ax1, bx1)
    yt = jnp.maximum(ay1, by1)
    xr = jnp.minimum(ax2, bx2)
    yb = jnp.minimum(ay2, by2)
    inter = (xr - xl) * (yb - yt)
    inter = jnp.where((xr >= xl) & (yb >= yt), inter, 0.0)
    area_a = (ax2 - ax1) * (ay2 - ay1)
    area_b = (bx2 - bx1) * (by2 - by1)
    iou = inter / (area_a + area_b - inter)

    best_iou = jnp.max(iou, axis=1, keepdims=True)
    col = jax.lax.broadcasted_iota(jnp.int32, iou.shape, 1)
    best_j = jnp.min(jnp.where(iou == best_iou, col, jnp.int32(1 << 30)),
                     axis=1, keepdims=True)
    has_match = best_iou > 0.0
    ig = 1.0 - best_iou

    outA_ref[...] = ig * sc_ref[...]

    onehot = col == best_j
    sp_best = jnp.sum(jnp.where(onehot, spr_ref[...], 0.0),
                      axis=1, keepdims=True)
    outBv_ref[...] = jnp.where(has_match, ig * sp_best, 0.0)
    gathered = [jnp.sum(jnp.where(onehot, bpT_ref[c:c + 1, :], 0.0),
                        axis=1, keepdims=True) for c in range(4)]
    outBb_ref[...] = jnp.concatenate(gathered, axis=1)

    contrib = jnp.sum(jnp.where(onehot & has_match, 1.0, 0.0),
                      axis=0, keepdims=True)

    @pl.when(step == 0)
    def _init():
        macc_ref[...] = jnp.zeros_like(macc_ref)

    macc_ref[...] = macc_ref[...] + contrib

    outC_ref[...] = jnp.where(macc_ref[...] > 0.0, 0.0, spr_ref[...])


def _paint_body(y1_ref, y2_ref, x1_ref, x2_ref, v_ref, out_ref, acc_ref):
    ybk = pl.program_id(0)
    kc = pl.program_id(1)

    @pl.when(kc == 0)
    def _init():
        acc_ref[...] = jnp.zeros_like(acc_ref)

    ys = (jax.lax.broadcasted_iota(jnp.int32, (1, 8), 1).astype(jnp.float32)
          + jnp.float32(8) * ybk.astype(jnp.float32))
    in_y = (ys >= y1_ref[...]) & (ys < y2_ref[...])
    a = jnp.where(in_y, v_ref[...], 0.0)
    xs = jax.lax.broadcasted_iota(jnp.int32, (_KC, 256), 1).astype(jnp.float32)
    fx = ((xs >= x1_ref[...]) & (xs < x2_ref[...])).astype(jnp.float32)
    t = a[:, :, None] * fx[:, None, :]
    acc_ref[...] = jnp.maximum(acc_ref[...], jnp.max(t, axis=0))

    @pl.when(kc == pl.num_programs(1) - 1)
    def _emit():
        m = acc_ref[:, :_WS]
        i1 = jax.lax.broadcasted_iota(jnp.int32, (_WS, 2 * _WS), 0)
        j1 = jax.lax.broadcasted_iota(jnp.int32, (_WS, 2 * _WS), 1)
        ex = ((j1 >> 1) == i1).astype(jnp.float32)
        i2 = jax.lax.broadcasted_iota(jnp.int32, (16, 8), 0)
        j2 = jax.lax.broadcasted_iota(jnp.int32, (16, 8), 1)
        dup = ((i2 >> 1) == j2).astype(jnp.float32)
        tmp = jax.lax.dot(m, ex, precision=jax.lax.Precision.HIGHEST,
                          preferred_element_type=jnp.float32)
        out_ref[...] = jax.lax.dot(dup, tmp,
                                   precision=jax.lax.Precision.HIGHEST,
                                   preferred_element_type=jnp.float32)


def kernel(inputs, boxes, scores, boxes_prev, scores_prev):
    n = boxes.shape[0]
    m = boxes_prev.shape[0]
    b = jnp.floor(boxes / _SUB)
    bp = jnp.floor(boxes_prev / _SUB)
    b_pad = jnp.zeros((_NPAD, 4), jnp.float32).at[:n].set(b)
    bp_pad = jnp.zeros((_NPAD, 4), jnp.float32).at[:m].set(bp)
    sc_pad = jnp.zeros((_NPAD, 1), jnp.float32).at[:n, 0].set(scores)
    spr_row = jnp.zeros((1, _NPAD), jnp.float32).at[0, :m].set(scores_prev)
    bpT = bp_pad.T

    steps = _NPAD // _RB
    outA, outBv, outBb, outC = pl.pallas_call(
        _match_body,
        grid=(steps,),
        in_specs=[
            pl.BlockSpec((_RB, 4), lambda i: (i, 0)),
            pl.BlockSpec((4, _NPAD), lambda i: (0, 0)),
            pl.BlockSpec((_RB, 1), lambda i: (i, 0)),
            pl.BlockSpec((1, _NPAD), lambda i: (0, 0)),
        ],
        out_specs=[
            pl.BlockSpec((_RB, 1), lambda i: (i, 0)),
            pl.BlockSpec((_RB, 1), lambda i: (i, 0)),
            pl.BlockSpec((_RB, 4), lambda i: (i, 0)),
            pl.BlockSpec((1, _NPAD), lambda i: (0, 0)),
        ],
        out_shape=[
            jax.ShapeDtypeStruct((_NPAD, 1), jnp.float32),
            jax.ShapeDtypeStruct((_NPAD, 1), jnp.float32),
            jax.ShapeDtypeStruct((_NPAD, 4), jnp.float32),
            jax.ShapeDtypeStruct((1, _NPAD), jnp.float32),
        ],
        scratch_shapes=[pltpu.VMEM((1, _NPAD), jnp.float32)],
    )(b_pad, bpT, sc_pad, spr_row)

    ev_boxes = jnp.concatenate([b_pad, outBb, bp_pad], axis=0)
    ev_vals = jnp.concatenate([outA, outBv, outC.T], axis=0)
    ev_x1 = ev_boxes[:, 0:1]
    ev_y1 = ev_boxes[:, 1:2]
    ev_x2 = ev_boxes[:, 2:3]
    ev_y2 = ev_boxes[:, 3:4]

    mask = pl.pallas_call(
        _paint_body,
        grid=(_HS // 8, _NEV // _KC),
        in_specs=[
            pl.BlockSpec((_KC, 1), lambda yb, kc: (kc, 0)),
            pl.BlockSpec((_KC, 1), lambda yb, kc: (kc, 0)),
            pl.BlockSpec((_KC, 1), lambda yb, kc: (kc, 0)),
            pl.BlockSpec((_KC, 1), lambda yb, kc: (kc, 0)),
            pl.BlockSpec((_KC, 1), lambda yb, kc: (kc, 0)),
        ],
        out_specs=pl.BlockSpec((16, 2 * _WS), lambda yb, kc: (yb, 0)),
        out_shape=jax.ShapeDtypeStruct((_H, _W), jnp.float32),
        scratch_shapes=[pltpu.VMEM((8, 256), jnp.float32)],
    )(ev_y1, ev_y2, ev_x1, ev_x2, ev_vals)

    return mask[None, None, :, :]


# speedup vs baseline: 2.1369x; 2.1369x over previous
import functools

import jax
import jax.numpy as jnp
from jax import lax
from jax.experimental import pallas as pl
from jax.experimental.pallas import tpu as pltpu
from jax.experimental.pallas import tpu_sc as plsc

_H = 384
_W = 384
_SUB = 2
_HS = _H // _SUB
_WS = _W // _SUB
_NPAD = 1024
_RB = 256
_NEV = 3 * _NPAD

_NC, _NS = 2, 16
_NW = _NC * _NS
_NACT = 24
_BAND = _HS // _NACT


def _match_body(b_ref, bpT_ref, sc_ref, spr_ref,
                outA_ref, outBv_ref, outBb_ref, outC_ref, macc_ref):
    step = pl.program_id(0)
    rows = b_ref[...]
    ax1 = rows[:, 0:1]
    ay1 = rows[:, 1:2]
    ax2 = rows[:, 2:3]
    ay2 = rows[:, 3:4]
    bx1 = bpT_ref[0:1, :]
    by1 = bpT_ref[1:2, :]
    bx2 = bpT_ref[2:3, :]
    by2 = bpT_ref[3:4, :]
    xl = jnp.maximum(ax1, bx1)
    yt = jnp.maximum(ay1, by1)
    xr = jnp.minimum(ax2, bx2)
    yb = jnp.minimum(ay2, by2)
    inter = (xr - xl) * (yb - yt)
    inter = jnp.where((xr >= xl) & (yb >= yt), inter, 0.0)
    area_a = (ax2 - ax1) * (ay2 - ay1)
    area_b = (bx2 - bx1) * (by2 - by1)
    iou = inter / (area_a + area_b - inter)

    best_iou = jnp.max(iou, axis=1, keepdims=True)
    col = jax.lax.broadcasted_iota(jnp.int32, iou.shape, 1)
    best_j = jnp.min(jnp.where(iou == best_iou, col, jnp.int32(1 << 30)),
                     axis=1, keepdims=True)
    has_match = best_iou > 0.0
    ig = 1.0 - best_iou

    outA_ref[...] = ig * sc_ref[...]

    onehot = col == best_j
    sp_best = jnp.sum(jnp.where(onehot, spr_ref[...], 0.0),
                      axis=1, keepdims=True)
    outBv_ref[...] = jnp.where(has_match, ig * sp_best, 0.0)
    gathered = [jnp.sum(jnp.where(onehot, bpT_ref[c:c + 1, :], 0.0),
                        axis=1, keepdims=True) for c in range(4)]
    outBb_ref[...] = jnp.concatenate(gathered, axis=1)

    contrib = jnp.sum(jnp.where(onehot & has_match, 1.0, 0.0),
                      axis=0, keepdims=True)

    @pl.when(step == 0)
    def _init():
        macc_ref[...] = jnp.zeros_like(macc_ref)

    macc_ref[...] = macc_ref[...] + contrib

    outC_ref[...] = jnp.where(macc_ref[...] > 0.0, 0.0, spr_ref[...])


def _paint_sc(ev_hbm, out_hbm, ev_v, bin_v, band_v, outb_v):
    cax = lax.axis_index("c")
    sax = lax.axis_index("s")
    wid = sax * _NC + cax

    @pl.when(wid < _NACT)
    def _active():
        _paint_band(wid, ev_hbm, out_hbm, ev_v, bin_v, band_v, outb_v)


def _paint_band(wid, ev_hbm, out_hbm, ev_v, bin_v, band_v, outb_v):
    lo_i = (wid * _BAND).astype(jnp.int32)
    lo_f = lo_i.astype(jnp.float32)
    hi_f = lo_f + jnp.float32(_BAND)

    pltpu.sync_copy(ev_hbm, ev_v)

    iota = lax.iota(jnp.int32, 16)
    zeros16 = jnp.zeros((16,), jnp.float32)
    for off in range(0, _BAND * _WS, 16):
        band_v[pl.ds(off, 16)] = zeros16

    def scan_body(i, cnt):
        off = i * 16
        y1 = ev_v[pl.ds(off, 16)]
        y2 = ev_v[pl.ds(_NEV + off, 16)]
        vv = ev_v[pl.ds(4 * _NEV + off, 16)]
        m = (y1 < hi_f) & (y2 > lo_f) & (vv > 0.0)
        mi = m.astype(jnp.int32)
        incl = plsc.cumsum(mi)
        plsc.store_scatter(bin_v, [cnt + incl - mi], off + iota, mask=m)
        return cnt + incl[15]

    total = lax.fori_loop(0, _NEV // 16, scan_body, jnp.int32(0))

    row_sel = jnp.minimum(iota, 4) * _NEV

    def ev_body(j, carry):
        e = bin_v[pl.ds(j, 16)][0]
        fields = plsc.load_gather(ev_v, [row_sel + e])
        y1 = fields[0]
        y2 = fields[1]
        x1 = fields[2]
        x2 = fields[3]
        val = fields[4]
        r0 = jnp.maximum(y1, lo_f).astype(jnp.int32) - lo_i
        r1 = jnp.minimum(y2, hi_f).astype(jnp.int32) - lo_i
        c0 = x1.astype(jnp.int32)
        c1 = x2.astype(jnp.int32)
        ch0 = c0 // 16
        ch1 = (c1 - 1) // 16 + 1
        vs = jnp.full((16,), val, jnp.float32)

        def row_body(r, rc):
            rb = r * _WS

            def ch_body(ch, cc):
                base = ch * 16
                pos = base + iota
                m = (pos >= c0) & (pos < c1)
                cur = band_v[pl.ds(rb + base, 16)]
                band_v[pl.ds(rb + base, 16)] = jnp.where(
                    m, jnp.maximum(cur, vs), cur)
                return cc

            return lax.fori_loop(ch0, ch1, ch_body, rc)

        return lax.fori_loop(r0, r1, row_body, carry)

    lax.fori_loop(0, total, ev_body, jnp.int32(0))

    half = iota >> 1
    for r in range(_BAND):
        for ch in range(_W // 16):
            g = plsc.load_gather(band_v, [r * _WS + 8 * ch + half])
            outb_v[pl.ds(2 * r * _W + 16 * ch, 16)] = g
            outb_v[pl.ds((2 * r + 1) * _W + 16 * ch, 16)] = g
    pltpu.sync_copy(outb_v,
                    out_hbm.at[pl.ds(wid * 2 * _BAND * _W, 2 * _BAND * _W)])


_paint_sc_call = functools.partial(
    pl.kernel,
    out_type=jax.ShapeDtypeStruct((_H * _W,), jnp.float32),
    mesh=plsc.VectorSubcoreMesh(core_axis_name="c", subcore_axis_name="s",
                                num_cores=_NC, num_subcores=_NS),
    scratch_types=[
        pltpu.VMEM((5 * _NEV,), jnp.float32),
        pltpu.VMEM((_NEV + 16,), jnp.int32),
        pltpu.VMEM((_BAND * _WS,), jnp.float32),
        pltpu.VMEM((2 * _BAND * _W,), jnp.float32),
    ],
    compiler_params=pltpu.CompilerParams(needs_layout_passes=False),
)(_paint_sc)


def kernel(inputs, boxes, scores, boxes_prev, scores_prev):
    n = boxes.shape[0]
    m = boxes_prev.shape[0]
    b = jnp.floor(boxes / _SUB)
    bp = jnp.floor(boxes_prev / _SUB)
    b_pad = jnp.zeros((_NPAD, 4), jnp.float32).at[:n].set(b)
    bp_pad = jnp.zeros((_NPAD, 4), jnp.float32).at[:m].set(bp)
    sc_pad = jnp.zeros((_NPAD, 1), jnp.float32).at[:n, 0].set(scores)
    spr_row = jnp.zeros((1, _NPAD), jnp.float32).at[0, :m].set(scores_prev)
    bpT = bp_pad.T

    steps = _NPAD // _RB
    outA, outBv, outBb, outC = pl.pallas_call(
        _match_body,
        grid=(steps,),
        in_specs=[
            pl.BlockSpec((_RB, 4), lambda i: (i, 0)),
            pl.BlockSpec((4, _NPAD), lambda i: (0, 0)),
            pl.BlockSpec((_RB, 1), lambda i: (i, 0)),
            pl.BlockSpec((1, _NPAD), lambda i: (0, 0)),
        ],
        out_specs=[
            pl.BlockSpec((_RB, 1), lambda i: (i, 0)),
            pl.BlockSpec((_RB, 1), lambda i: (i, 0)),
            pl.BlockSpec((_RB, 4), lambda i: (i, 0)),
            pl.BlockSpec((1, _NPAD), lambda i: (0, 0)),
        ],
        out_shape=[
            jax.ShapeDtypeStruct((_NPAD, 1), jnp.float32),
            jax.ShapeDtypeStruct((_NPAD, 1), jnp.float32),
            jax.ShapeDtypeStruct((_NPAD, 4), jnp.float32),
            jax.ShapeDtypeStruct((1, _NPAD), jnp.float32),
        ],
        scratch_shapes=[pltpu.VMEM((1, _NPAD), jnp.float32)],
    )(b_pad, bpT, sc_pad, spr_row)

    ev_boxes = jnp.concatenate([b_pad, outBb, bp_pad], axis=0)
    ev_vals = jnp.concatenate([outA[:, 0], outBv[:, 0], outC[0, :]])
    ev = jnp.concatenate([ev_boxes[:, 1], ev_boxes[:, 3],
                          ev_boxes[:, 0], ev_boxes[:, 2], ev_vals])

    mask = _paint_sc_call(ev)
    return mask.reshape(1, 1, _H, _W)
